# baseline (device time: 12097 ns/iter reference)
import jax
import jax.numpy as jnp
from jax import lax
from jax.experimental import pallas as pl
from jax.experimental.pallas import tpu as pltpu

N_DEV = 16


def kernel(x, pi):
    m, n = x.shape[1], x.shape[2]
    x = pltpu.with_memory_space_constraint(x, pltpu.MemorySpace.HBM)
    pi = pltpu.with_memory_space_constraint(pi, pltpu.MemorySpace.HBM)
    dummy = jnp.zeros((1, m, n), jnp.bfloat16)
    dummy = pltpu.with_memory_space_constraint(dummy, pltpu.MemorySpace.HBM)

    def body(
        x_hbm, pi_hbm, dummy_hbm, out_hbm,
        xv, send_buf, pi_smem, load_sem, pi_sem, send_sem, recv_sem,
    ):
        my = lax.axis_index("i")

        load = pltpu.make_async_copy(x_hbm, xv, load_sem)
        load.start()
        pi_load = pltpu.make_async_copy(pi_hbm, pi_smem, pi_sem)
        pi_load.start()

        barrier_sem = pltpu.get_barrier_semaphore()
        for peer in range(N_DEV):
            pl.semaphore_signal(
                barrier_sem, inc=1,
                device_id=(peer,), device_id_type=pl.DeviceIdType.MESH,
            )

        load.wait()
        send_buf[...] = xv[...].astype(jnp.bfloat16)
        pi_load.wait()
        dst = pi_smem[my]

        pl.semaphore_wait(barrier_sem, N_DEV)
        rdma = pltpu.make_async_remote_copy(
            src_ref=send_buf,
            dst_ref=out_hbm,
            send_sem=send_sem,
            recv_sem=recv_sem,
            device_id=(dst,),
            device_id_type=pl.DeviceIdType.MESH,
        )
        rdma.start()
        rdma.wait()

    return pl.pallas_call(
        body,
        out_shape=jax.ShapeDtypeStruct(x.shape, jnp.bfloat16),
        in_specs=[
            pl.BlockSpec(memory_space=pltpu.MemorySpace.HBM),
            pl.BlockSpec(memory_space=pltpu.MemorySpace.HBM),
            pl.BlockSpec(memory_space=pltpu.MemorySpace.HBM),
        ],
        out_specs=pl.BlockSpec(memory_space=pltpu.MemorySpace.HBM),
        input_output_aliases={2: 0},
        scratch_shapes=[
            pltpu.VMEM((1, m, n), jnp.float32),
            pltpu.VMEM((1, m, n), jnp.bfloat16),
            pltpu.SMEM((N_DEV,), jnp.int32),
            pltpu.SemaphoreType.DMA,
            pltpu.SemaphoreType.DMA,
            pltpu.SemaphoreType.DMA,
            pltpu.SemaphoreType.DMA,
        ],
        compiler_params=pltpu.CompilerParams(collective_id=0),
    )(x, pi, dummy)


# device time: 10386 ns/iter; 1.1647x vs baseline; 1.1647x over previous
import jax
import jax.numpy as jnp
from jax import lax
from jax.experimental import pallas as pl
from jax.experimental.pallas import tpu as pltpu

N_DEV = 16


def kernel(x, pi):
    m, n = x.shape[1], x.shape[2]
    x = pltpu.with_memory_space_constraint(x, pltpu.MemorySpace.HBM)
    pi = pltpu.with_memory_space_constraint(pi, pltpu.MemorySpace.HBM)

    def body(
        x_hbm, pi_hbm, out_hbm,
        xv, send_buf, pi_smem, load_sem, pi_sem, send_sem, recv_sem,
    ):
        my = lax.axis_index("i")

        load = pltpu.make_async_copy(x_hbm, xv, load_sem)
        load.start()
        pi_load = pltpu.make_async_copy(pi_hbm, pi_smem, pi_sem)
        pi_load.start()

        barrier_sem = pltpu.get_barrier_semaphore()
        for peer in range(N_DEV):
            pl.semaphore_signal(
                barrier_sem, inc=1,
                device_id=(peer,), device_id_type=pl.DeviceIdType.MESH,
            )

        load.wait()
        send_buf[...] = xv[...].astype(jnp.bfloat16)
        pi_load.wait()
        dst = pi_smem[my]

        pl.semaphore_wait(barrier_sem, N_DEV)
        rdma = pltpu.make_async_remote_copy(
            src_ref=send_buf,
            dst_ref=out_hbm,
            send_sem=send_sem,
            recv_sem=recv_sem,
            device_id=(dst,),
            device_id_type=pl.DeviceIdType.MESH,
        )
        rdma.start()
        rdma.wait()

    return pl.pallas_call(
        body,
        out_shape=jax.ShapeDtypeStruct(x.shape, jnp.bfloat16),
        in_specs=[
            pl.BlockSpec(memory_space=pltpu.MemorySpace.HBM),
            pl.BlockSpec(memory_space=pltpu.MemorySpace.HBM),
        ],
        out_specs=pl.BlockSpec(memory_space=pltpu.MemorySpace.HBM),
        scratch_shapes=[
            pltpu.VMEM((1, m, n), jnp.float32),
            pltpu.VMEM((1, m, n), jnp.bfloat16),
            pltpu.SMEM((N_DEV,), jnp.int32),
            pltpu.SemaphoreType.DMA,
            pltpu.SemaphoreType.DMA,
            pltpu.SemaphoreType.DMA,
            pltpu.SemaphoreType.DMA,
        ],
        compiler_params=pltpu.CompilerParams(collective_id=0),
    )(x, pi)
